# baseline (speedup 1.0000x reference)
import functools
import math

import jax
import jax.numpy as jnp
from jax import lax
from jax.experimental import pallas as pl
from jax.experimental.pallas import tpu as pltpu
from jax.experimental.pallas import tpu_sc as plsc

N_NODES = 50000
N_EDGES = 800000
HID = 64
N_GRAPHS = 64

NC = 2
NS = 16
R = 25024
NPAD = NC * R
ACC_ROWS = 25160
TRASH = R
CHUNK = 128
IDXCH = 12
SUP = 3
NIG = 33
NCH = IDXCH * NIG
EPAD = NS * NCH * CHUNK
EROWS = EPAD // CHUNK
NCHP = NCH + IDXCH
LROWS = NS * NCHP
ZROWS = R // NS

_BN_S = 1.0 / math.sqrt(1.0 + 1e-5)


def _sc_mesh():
    return plsc.VectorSubcoreMesh(core_axis_name="c", subcore_axis_name="s")


def _remap_dst(dstb, base):
    for r in range(IDXCH):
        def remap(i, _, r=r):
            c = i * 16
            d = dstb[r, pl.ds(c, 16)]
            ok = (d >= base) & (d < base + R)
            dstb[r, pl.ds(c, 16)] = jnp.where(ok, d - base,
                                              TRASH + (d & 127))
            return 0

        lax.fori_loop(0, CHUNK // 16, remap, 0)


def _copy_out_stripes(sid, base, srcs_dsts):
    big = 1568

    @pl.when(sid < NS - 1)
    def _copy_big():
        for s, d in srcs_dsts:
            pltpu.sync_copy(s.at[pl.ds(sid * big, big)],
                            d.at[pl.ds(base + sid * big, big)])

    @pl.when(sid == NS - 1)
    def _copy_last():
        last = R - (NS - 1) * big
        for s, d in srcs_dsts:
            pltpu.sync_copy(s.at[pl.ds((NS - 1) * big, last)],
                            d.at[pl.ds(base + (NS - 1) * big, last)])


def _sc_spmv_body(h_hbm, src2d_hbm, dstloc_hbm, z64_hbm, ah_out,
                  srcb, dstb, rows, acc, gsem, ssem):
    cid = lax.axis_index("c")
    sid = lax.axis_index("s")
    base = cid * R

    pltpu.sync_copy(z64_hbm, acc.at[pl.ds(sid * ZROWS, ZROWS)])
    plsc.subcore_barrier()

    def group_body(g, _):
        row0 = sid * NCH + g * IDXCH
        pltpu.sync_copy(src2d_hbm.at[pl.ds(row0, IDXCH)], srcb)
        pltpu.sync_copy(dstloc_hbm.at[cid, pl.ds(row0, IDXCH)], dstb)
        gd = [None] * IDXCH
        sd = [None] * IDXCH
        for j in range(IDXCH):
            if j >= SUP:
                sd[j - SUP].wait()
            gd[j] = pltpu.async_copy(h_hbm.at[srcb.at[j]],
                                     rows.at[j % SUP], gsem)
            if j >= 1:
                gd[j - 1].wait()
                sd[j - 1] = pltpu.async_copy(rows.at[(j - 1) % SUP],
                                             acc.at[dstb.at[j - 1]], ssem,
                                             add=True)
        gd[IDXCH - 1].wait()
        sd[IDXCH - 1] = pltpu.async_copy(rows.at[(IDXCH - 1) % SUP],
                                         acc.at[dstb.at[IDXCH - 1]], ssem,
                                         add=True)
        for j in range(IDXCH - SUP, IDXCH):
            sd[j].wait()
        return 0

    lax.fori_loop(0, NIG, group_body, 0)
    plsc.subcore_barrier()
    _copy_out_stripes(sid, base, [(acc, ah_out)])


_sc_spmv = pl.kernel(
    _sc_spmv_body,
    mesh=_sc_mesh(),
    out_type=jax.ShapeDtypeStruct((NPAD, HID), jnp.float32),
    scratch_types=[
        pltpu.VMEM((IDXCH, CHUNK), jnp.int32),
        pltpu.VMEM((IDXCH, CHUNK), jnp.int32),
        pltpu.VMEM((SUP, CHUNK, HID), jnp.float32),
        pltpu.VMEM_SHARED((ACC_ROWS, HID), jnp.float32),
        pltpu.SemaphoreType.DMA,
        pltpu.SemaphoreType.DMA,
    ],
    compiler_params=pltpu.CompilerParams(use_tc_tiling_on_sc=False),
)


def _sc_deg_body(dst2d_hbm, z16_hbm, one16_hbm, deg_out, dstloc_out,
                 dstb, onesb, dacc, ssem):
    cid = lax.axis_index("c")
    sid = lax.axis_index("s")
    base = cid * R

    pltpu.sync_copy(z16_hbm, dacc.at[pl.ds(sid * ZROWS, ZROWS)])
    pltpu.sync_copy(one16_hbm, onesb)
    plsc.subcore_barrier()

    def group_body(g, _):
        row0 = sid * NCH + g * IDXCH
        pltpu.sync_copy(dst2d_hbm.at[pl.ds(row0, IDXCH)], dstb)
        _remap_dst(dstb, base)
        pltpu.sync_copy(dstb, dstloc_out.at[cid, pl.ds(row0, IDXCH)])
        sd = [pltpu.async_copy(onesb, dacc.at[dstb.at[b]], ssem, add=True)
              for b in range(IDXCH)]
        for d in sd:
            d.wait()
        return 0

    lax.fori_loop(0, NIG, group_body, 0)
    plsc.subcore_barrier()
    _copy_out_stripes(sid, base, [(dacc, deg_out)])


_sc_deg = pl.kernel(
    _sc_deg_body,
    mesh=_sc_mesh(),
    out_type=[
        jax.ShapeDtypeStruct((NPAD, 16), jnp.float32),
        jax.ShapeDtypeStruct((NC, EROWS, CHUNK), jnp.int32),
    ],
    scratch_types=[
        pltpu.VMEM((IDXCH, CHUNK), jnp.int32),
        pltpu.VMEM((CHUNK, 16), jnp.float32),
        pltpu.VMEM_SHARED((ACC_ROWS, 16), jnp.float32),
        pltpu.SemaphoreType.DMA,
    ],
    compiler_params=pltpu.CompilerParams(use_tc_tiling_on_sc=False),
)


NB = 3128
NBLK = NPAD // NB


def _tc_dense1_body(h_ref, w_ref, b_ref, a_ref, be_ref, o_ref):
    z = jnp.dot(h_ref[...], w_ref[...], preferred_element_type=jnp.float32)
    z = jax.nn.relu(z + b_ref[...])
    o_ref[...] = z * a_ref[...] + be_ref[...]


def _tc_dense1(hp, wT, b, a, be):
    return pl.pallas_call(
        _tc_dense1_body,
        grid=(NBLK,),
        in_specs=[
            pl.BlockSpec((NB, hp.shape[1]), lambda i: (i, 0)),
            pl.BlockSpec(wT.shape, lambda i: (0, 0)),
            pl.BlockSpec((1, HID), lambda i: (0, 0)),
            pl.BlockSpec((1, HID), lambda i: (0, 0)),
            pl.BlockSpec((1, HID), lambda i: (0, 0)),
        ],
        out_specs=pl.BlockSpec((NB, HID), lambda i: (i, 0)),
        out_shape=jax.ShapeDtypeStruct((NPAD, HID), jnp.float32),
    )(hp, wT, b, a, be)


def _tc_mid_body(t_ref, ah_ref, deg_ref, w2a_ref, w2b_ref, b2_ref, a2_ref,
                 be2_ref, w1_ref, b1_ref, a1_ref, be1_ref, o_ref):
    t = t_ref[...]
    deg = deg_ref[...][:, 0:1]
    u = jnp.dot(t * deg, w2a_ref[...], preferred_element_type=jnp.float32)
    u = u + jnp.dot(ah_ref[...], w2b_ref[...],
                    preferred_element_type=jnp.float32)
    u = jax.nn.relu(u + b2_ref[...])
    z = u * a2_ref[...] + be2_ref[...]
    r = jax.nn.relu(z)
    v = jnp.dot(r, w1_ref[...], preferred_element_type=jnp.float32)
    v = jax.nn.relu(v + b1_ref[...])
    o_ref[...] = v * a1_ref[...] + be1_ref[...]


def _tc_mid(t, ah, deg16, w2aT, w2bT, b2, a2, be2, w1T, b1, a1, be1):
    vec = pl.BlockSpec((1, HID), lambda i: (0, 0))
    return pl.pallas_call(
        _tc_mid_body,
        grid=(NBLK,),
        in_specs=[
            pl.BlockSpec((NB, HID), lambda i: (i, 0)),
            pl.BlockSpec((NB, HID), lambda i: (i, 0)),
            pl.BlockSpec((NB, 16), lambda i: (i, 0)),
            pl.BlockSpec((HID, HID), lambda i: (0, 0)),
            pl.BlockSpec((HID, HID), lambda i: (0, 0)),
            vec, vec, vec,
            pl.BlockSpec((HID, HID), lambda i: (0, 0)),
            vec, vec, vec,
        ],
        out_specs=pl.BlockSpec((NB, HID), lambda i: (i, 0)),
        out_shape=jax.ShapeDtypeStruct((NPAD, HID), jnp.float32),
    )(t, ah, deg16, w2aT, w2bT, b2, a2, be2, w1T, b1, a1, be1)


def _tc_final_body(t_ref, ah_ref, deg_ref, batch_ref, w2a_ref, w2b_ref,
                   b2_ref, a2_ref, be2_ref, wc1_ref, bc1_ref, ac_ref,
                   bec_ref, wc2_ref, bc2_ref, o_ref, pool_acc, cnt_acc):
    i = pl.program_id(0)

    @pl.when(i == 0)
    def _init():
        pool_acc[...] = jnp.zeros_like(pool_acc)
        cnt_acc[...] = jnp.zeros_like(cnt_acc)

    t = t_ref[...]
    deg = deg_ref[...][:, 0:1]
    u = jnp.dot(t * deg, w2a_ref[...], preferred_element_type=jnp.float32)
    u = u + jnp.dot(ah_ref[...], w2b_ref[...],
                    preferred_element_type=jnp.float32)
    u = jax.nn.relu(u + b2_ref[...])
    u = u * a2_ref[...] + be2_ref[...]

    gids = lax.broadcasted_iota(jnp.int32, (1, N_GRAPHS), 1)
    oh = (batch_ref[...] == gids).astype(jnp.float32)
    dn = (((0,), (0,)), ((), ()))
    pool_acc[...] += lax.dot_general(oh, u, dn,
                                     preferred_element_type=jnp.float32)
    ones = jnp.ones((NB, 8), jnp.float32)
    cnt_acc[...] += lax.dot_general(oh, ones, dn,
                                    preferred_element_type=jnp.float32)

    @pl.when(i == NBLK - 1)
    def _finish():
        cnt = jnp.maximum(cnt_acc[...][:, 0:1], 1.0)
        pooled = pool_acc[...] / cnt
        c = jnp.dot(pooled, wc1_ref[...], preferred_element_type=jnp.float32)
        c = jax.nn.relu(c + bc1_ref[...])
        c = c * ac_ref[...] + bec_ref[...]
        lg = jnp.dot(c, wc2_ref[...], preferred_element_type=jnp.float32)
        lg = lg + bc2_ref[...]
        m = jnp.max(lg, axis=1, keepdims=True)
        lse = m + jnp.log(jnp.sum(jnp.exp(lg - m), axis=1, keepdims=True))
        o_ref[...] = lg - lse


def _tc_final(t, ah, deg16, batch2d, w2aT, w2bT, b2, a2, be2,
              wc1T, bc1, ac, bec, wc2T, bc2):
    vec = pl.BlockSpec((1, HID), lambda i: (0, 0))
    return pl.pallas_call(
        _tc_final_body,
        grid=(NBLK,),
        in_specs=[
            pl.BlockSpec((NB, HID), lambda i: (i, 0)),
            pl.BlockSpec((NB, HID), lambda i: (i, 0)),
            pl.BlockSpec((NB, 16), lambda i: (i, 0)),
            pl.BlockSpec((NB, 1), lambda i: (i, 0)),
            pl.BlockSpec((HID, HID), lambda i: (0, 0)),
            pl.BlockSpec((HID, HID), lambda i: (0, 0)),
            vec, vec, vec,
            pl.BlockSpec((HID, HID), lambda i: (0, 0)),
            vec, vec, vec,
            pl.BlockSpec((HID, 16), lambda i: (0, 0)),
            pl.BlockSpec((1, 16), lambda i: (0, 0)),
        ],
        out_specs=pl.BlockSpec((N_GRAPHS, 16), lambda i: (0, 0)),
        out_shape=jax.ShapeDtypeStruct((N_GRAPHS, 16), jnp.float32),
        scratch_shapes=[
            pltpu.VMEM((N_GRAPHS, HID), jnp.float32),
            pltpu.VMEM((N_GRAPHS, 8), jnp.float32),
        ],
    )(t, ah, deg16, batch2d, w2aT, w2bT, b2, a2, be2,
      wc1T, bc1, ac, bec, wc2T, bc2)


def _row(v):
    return jnp.reshape(v, (1, -1)).astype(jnp.float32)


def kernel(x, pos_enc, edge_index, batch, params):
    p = params
    f32 = jnp.float32

    h0 = jnp.concatenate([x, pos_enc], axis=-1).astype(f32)
    h0p = jnp.zeros((NPAD, 16), f32).at[:N_NODES, :h0.shape[1]].set(h0)

    src = edge_index[0].astype(jnp.int32)
    dst = edge_index[1].astype(jnp.int32)
    pad_e = EPAD - N_EDGES
    srcp = jnp.concatenate(
        [src, jnp.zeros((pad_e,), jnp.int32)]).reshape(EROWS, CHUNK)
    dstp = jnp.concatenate(
        [dst, jnp.full((pad_e,), 1 << 29, jnp.int32)]).reshape(EROWS, CHUNK)

    batchp = jnp.concatenate(
        [batch.astype(jnp.int32), jnp.full((NPAD - N_NODES,), N_GRAPHS,
                                           jnp.int32)]).reshape(NPAD, 1)

    z64 = jnp.zeros((ZROWS, HID), f32)
    z16 = jnp.zeros((ZROWS, 16), f32)
    one16 = jnp.ones((CHUNK, 16), f32)

    def bn_vecs(pre, k):
        g = p[pre + "_g" + k] * _BN_S
        return (_row(p[pre + "_b" + k]), _row(g), _row(p[pre + "_be" + k]))

    w1c1 = jnp.zeros((16, HID), f32).at[:9, :].set(p["c1_W1"].T)
    b1, a1, be1 = bn_vecs("c1", "1")
    t1 = _tc_dense1(h0p, w1c1, b1, a1, be1)

    deg16, dstloc = _sc_deg(dstp, z16, one16)
    ah1 = _sc_spmv(t1, srcp, dstloc, z64)

    def mid(t, ah, pre_out, pre_in):
        w2 = p[pre_out + "_W2"]
        w2aT = w2[:, :HID].T.astype(f32)
        w2bT = w2[:, HID:].T.astype(f32)
        b2, a2, be2 = bn_vecs(pre_out, "2")
        w1T = p[pre_in + "_W1"].T.astype(f32)
        b1n, a1n, be1n = bn_vecs(pre_in, "1")
        return _tc_mid(t, ah, deg16, w2aT, w2bT, b2, a2, be2,
                       w1T, b1n, a1n, be1n)

    t2 = mid(t1, ah1, "c1", "c2")
    t3 = mid(t2, _sc_spmv(t2, srcp, dstloc, z64), "c2", "c3")
    ah3 = _sc_spmv(t3, srcp, dstloc, z64)

    w2 = p["c3_W2"]
    w2aT = w2[:, :HID].T.astype(f32)
    w2bT = w2[:, HID:].T.astype(f32)
    b2, a2, be2 = bn_vecs("c3", "2")
    wc1T = p["cls_W1"].T.astype(f32)
    bc1 = _row(p["cls_b1"])
    ac = _row(p["cls_g"] * _BN_S)
    bec = _row(p["cls_be"])
    wc2T = jnp.zeros((HID, 16), f32).at[:, :10].set(p["cls_W2"].T)
    bc2 = jnp.full((1, 16), -1e30, f32).at[0, :10].set(p["cls_b2"])

    out = _tc_final(t3, ah3, deg16, batchp, w2aT, w2bT, b2, a2, be2,
                    wc1T, bc1, ac, bec, wc2T, bc2)
    return out[:, :10]


# speedup vs baseline: 1.0149x; 1.0149x over previous
import functools
import math

import jax
import jax.numpy as jnp
from jax import lax
from jax.experimental import pallas as pl
from jax.experimental.pallas import tpu as pltpu
from jax.experimental.pallas import tpu_sc as plsc

N_NODES = 50000
N_EDGES = 800000
HID = 64
N_GRAPHS = 64

NC = 2
NS = 16
R = 25024
NPAD = NC * R
ACC_ROWS = 25160
TRASH = R
CHUNK = 128
IDXCH = 12
SUP = 3
NIG = 33
NCH = IDXCH * NIG
EPAD = NS * NCH * CHUNK
EROWS = EPAD // CHUNK
NCHP = NCH + IDXCH
LROWS = NS * NCHP
ZROWS = R // NS

_BN_S = 1.0 / math.sqrt(1.0 + 1e-5)


def _sc_mesh():
    return plsc.VectorSubcoreMesh(core_axis_name="c", subcore_axis_name="s")


def _remap_dst(dstb, base):
    for r in range(IDXCH):
        def remap(i, _, r=r):
            c = i * 16
            d = dstb[r, pl.ds(c, 16)]
            ok = (d >= base) & (d < base + R)
            dstb[r, pl.ds(c, 16)] = jnp.where(ok, d - base,
                                              TRASH + (d & 127))
            return 0

        lax.fori_loop(0, CHUNK // 16, remap, 0)


def _copy_out_stripes(sid, base, srcs_dsts):
    big = 1568

    @pl.when(sid < NS - 1)
    def _copy_big():
        for s, d in srcs_dsts:
            pltpu.sync_copy(s.at[pl.ds(sid * big, big)],
                            d.at[pl.ds(base + sid * big, big)])

    @pl.when(sid == NS - 1)
    def _copy_last():
        last = R - (NS - 1) * big
        for s, d in srcs_dsts:
            pltpu.sync_copy(s.at[pl.ds((NS - 1) * big, last)],
                            d.at[pl.ds(base + (NS - 1) * big, last)])


def _sc_spmv_body(h_hbm, src2d_hbm, dstloc_hbm, z64_hbm, ah_out,
                  srcb, dstb, rows, acc, gsem, ssem):
    cid = lax.axis_index("c")
    sid = lax.axis_index("s")
    base = cid * R

    pltpu.sync_copy(z64_hbm, acc.at[pl.ds(sid * ZROWS, ZROWS)])
    plsc.subcore_barrier()

    def group_body(g, _):
        row0 = sid * NCH + g * IDXCH
        ld0 = pltpu.async_copy(src2d_hbm.at[pl.ds(row0, IDXCH)], srcb,
                               gsem)
        ld1 = pltpu.async_copy(dstloc_hbm.at[cid, pl.ds(row0, IDXCH)],
                               dstb, gsem)
        ld0.wait()
        ld1.wait()
        gd = [None] * IDXCH
        sd = [None] * IDXCH
        for j in range(IDXCH):
            if j >= SUP:
                sd[j - SUP].wait()
            gd[j] = pltpu.async_copy(h_hbm.at[srcb.at[j]],
                                     rows.at[j % SUP], gsem)
            if j >= 1:
                gd[j - 1].wait()
                sd[j - 1] = pltpu.async_copy(rows.at[(j - 1) % SUP],
                                             acc.at[dstb.at[j - 1]], ssem,
                                             add=True)
        gd[IDXCH - 1].wait()
        sd[IDXCH - 1] = pltpu.async_copy(rows.at[(IDXCH - 1) % SUP],
                                         acc.at[dstb.at[IDXCH - 1]], ssem,
                                         add=True)
        for j in range(IDXCH - SUP, IDXCH):
            sd[j].wait()
        return 0

    lax.fori_loop(0, NIG, group_body, 0)
    plsc.subcore_barrier()
    _copy_out_stripes(sid, base, [(acc, ah_out)])


_sc_spmv = pl.kernel(
    _sc_spmv_body,
    mesh=_sc_mesh(),
    out_type=jax.ShapeDtypeStruct((NPAD, HID), jnp.float32),
    scratch_types=[
        pltpu.VMEM((IDXCH, CHUNK), jnp.int32),
        pltpu.VMEM((IDXCH, CHUNK), jnp.int32),
        pltpu.VMEM((SUP, CHUNK, HID), jnp.float32),
        pltpu.VMEM_SHARED((ACC_ROWS, HID), jnp.float32),
        pltpu.SemaphoreType.DMA,
        pltpu.SemaphoreType.DMA,
    ],
    compiler_params=pltpu.CompilerParams(use_tc_tiling_on_sc=False),
)


def _sc_deg_body(dst2d_hbm, z16_hbm, one16_hbm, deg_out, dstloc_out,
                 dstb, onesb, dacc, ssem):
    cid = lax.axis_index("c")
    sid = lax.axis_index("s")
    base = cid * R

    pltpu.sync_copy(z16_hbm, dacc.at[pl.ds(sid * ZROWS, ZROWS)])
    pltpu.sync_copy(one16_hbm, onesb)
    plsc.subcore_barrier()

    def group_body(g, _):
        row0 = sid * NCH + g * IDXCH
        pltpu.sync_copy(dst2d_hbm.at[pl.ds(row0, IDXCH)], dstb)
        _remap_dst(dstb, base)
        pltpu.sync_copy(dstb, dstloc_out.at[cid, pl.ds(row0, IDXCH)])
        sd = [pltpu.async_copy(onesb, dacc.at[dstb.at[b]], ssem, add=True)
              for b in range(IDXCH)]
        for d in sd:
            d.wait()
        return 0

    lax.fori_loop(0, NIG, group_body, 0)
    plsc.subcore_barrier()
    _copy_out_stripes(sid, base, [(dacc, deg_out)])


_sc_deg = pl.kernel(
    _sc_deg_body,
    mesh=_sc_mesh(),
    out_type=[
        jax.ShapeDtypeStruct((NPAD, 16), jnp.float32),
        jax.ShapeDtypeStruct((NC, EROWS, CHUNK), jnp.int32),
    ],
    scratch_types=[
        pltpu.VMEM((IDXCH, CHUNK), jnp.int32),
        pltpu.VMEM((CHUNK, 16), jnp.float32),
        pltpu.VMEM_SHARED((ACC_ROWS, 16), jnp.float32),
        pltpu.SemaphoreType.DMA,
    ],
    compiler_params=pltpu.CompilerParams(use_tc_tiling_on_sc=False),
)


NB = 3128
NBLK = NPAD // NB


def _tc_dense1_body(h_ref, w_ref, b_ref, a_ref, be_ref, o_ref):
    z = jnp.dot(h_ref[...], w_ref[...], preferred_element_type=jnp.float32)
    z = jax.nn.relu(z + b_ref[...])
    o_ref[...] = z * a_ref[...] + be_ref[...]


def _tc_dense1(hp, wT, b, a, be):
    return pl.pallas_call(
        _tc_dense1_body,
        grid=(NBLK,),
        in_specs=[
            pl.BlockSpec((NB, hp.shape[1]), lambda i: (i, 0)),
            pl.BlockSpec(wT.shape, lambda i: (0, 0)),
            pl.BlockSpec((1, HID), lambda i: (0, 0)),
            pl.BlockSpec((1, HID), lambda i: (0, 0)),
            pl.BlockSpec((1, HID), lambda i: (0, 0)),
        ],
        out_specs=pl.BlockSpec((NB, HID), lambda i: (i, 0)),
        out_shape=jax.ShapeDtypeStruct((NPAD, HID), jnp.float32),
    )(hp, wT, b, a, be)


def _tc_mid_body(t_ref, ah_ref, deg_ref, w2a_ref, w2b_ref, b2_ref, a2_ref,
                 be2_ref, w1_ref, b1_ref, a1_ref, be1_ref, o_ref):
    t = t_ref[...]
    deg = deg_ref[...][:, 0:1]
    u = jnp.dot(t * deg, w2a_ref[...], preferred_element_type=jnp.float32)
    u = u + jnp.dot(ah_ref[...], w2b_ref[...],
                    preferred_element_type=jnp.float32)
    u = jax.nn.relu(u + b2_ref[...])
    z = u * a2_ref[...] + be2_ref[...]
    r = jax.nn.relu(z)
    v = jnp.dot(r, w1_ref[...], preferred_element_type=jnp.float32)
    v = jax.nn.relu(v + b1_ref[...])
    o_ref[...] = v * a1_ref[...] + be1_ref[...]


def _tc_mid(t, ah, deg16, w2aT, w2bT, b2, a2, be2, w1T, b1, a1, be1):
    vec = pl.BlockSpec((1, HID), lambda i: (0, 0))
    return pl.pallas_call(
        _tc_mid_body,
        grid=(NBLK,),
        in_specs=[
            pl.BlockSpec((NB, HID), lambda i: (i, 0)),
            pl.BlockSpec((NB, HID), lambda i: (i, 0)),
            pl.BlockSpec((NB, 16), lambda i: (i, 0)),
            pl.BlockSpec((HID, HID), lambda i: (0, 0)),
            pl.BlockSpec((HID, HID), lambda i: (0, 0)),
            vec, vec, vec,
            pl.BlockSpec((HID, HID), lambda i: (0, 0)),
            vec, vec, vec,
        ],
        out_specs=pl.BlockSpec((NB, HID), lambda i: (i, 0)),
        out_shape=jax.ShapeDtypeStruct((NPAD, HID), jnp.float32),
    )(t, ah, deg16, w2aT, w2bT, b2, a2, be2, w1T, b1, a1, be1)


def _tc_final_body(t_ref, ah_ref, deg_ref, batch_ref, w2a_ref, w2b_ref,
                   b2_ref, a2_ref, be2_ref, wc1_ref, bc1_ref, ac_ref,
                   bec_ref, wc2_ref, bc2_ref, o_ref, pool_acc, cnt_acc):
    i = pl.program_id(0)

    @pl.when(i == 0)
    def _init():
        pool_acc[...] = jnp.zeros_like(pool_acc)
        cnt_acc[...] = jnp.zeros_like(cnt_acc)

    t = t_ref[...]
    deg = deg_ref[...][:, 0:1]
    u = jnp.dot(t * deg, w2a_ref[...], preferred_element_type=jnp.float32)
    u = u + jnp.dot(ah_ref[...], w2b_ref[...],
                    preferred_element_type=jnp.float32)
    u = jax.nn.relu(u + b2_ref[...])
    u = u * a2_ref[...] + be2_ref[...]

    gids = lax.broadcasted_iota(jnp.int32, (1, N_GRAPHS), 1)
    oh = (batch_ref[...] == gids).astype(jnp.float32)
    dn = (((0,), (0,)), ((), ()))
    pool_acc[...] += lax.dot_general(oh, u, dn,
                                     preferred_element_type=jnp.float32)
    ones = jnp.ones((NB, 8), jnp.float32)
    cnt_acc[...] += lax.dot_general(oh, ones, dn,
                                    preferred_element_type=jnp.float32)

    @pl.when(i == NBLK - 1)
    def _finish():
        cnt = jnp.maximum(cnt_acc[...][:, 0:1], 1.0)
        pooled = pool_acc[...] / cnt
        c = jnp.dot(pooled, wc1_ref[...], preferred_element_type=jnp.float32)
        c = jax.nn.relu(c + bc1_ref[...])
        c = c * ac_ref[...] + bec_ref[...]
        lg = jnp.dot(c, wc2_ref[...], preferred_element_type=jnp.float32)
        lg = lg + bc2_ref[...]
        m = jnp.max(lg, axis=1, keepdims=True)
        lse = m + jnp.log(jnp.sum(jnp.exp(lg - m), axis=1, keepdims=True))
        o_ref[...] = lg - lse


def _tc_final(t, ah, deg16, batch2d, w2aT, w2bT, b2, a2, be2,
              wc1T, bc1, ac, bec, wc2T, bc2):
    vec = pl.BlockSpec((1, HID), lambda i: (0, 0))
    return pl.pallas_call(
        _tc_final_body,
        grid=(NBLK,),
        in_specs=[
            pl.BlockSpec((NB, HID), lambda i: (i, 0)),
            pl.BlockSpec((NB, HID), lambda i: (i, 0)),
            pl.BlockSpec((NB, 16), lambda i: (i, 0)),
            pl.BlockSpec((NB, 1), lambda i: (i, 0)),
            pl.BlockSpec((HID, HID), lambda i: (0, 0)),
            pl.BlockSpec((HID, HID), lambda i: (0, 0)),
            vec, vec, vec,
            pl.BlockSpec((HID, HID), lambda i: (0, 0)),
            vec, vec, vec,
            pl.BlockSpec((HID, 16), lambda i: (0, 0)),
            pl.BlockSpec((1, 16), lambda i: (0, 0)),
        ],
        out_specs=pl.BlockSpec((N_GRAPHS, 16), lambda i: (0, 0)),
        out_shape=jax.ShapeDtypeStruct((N_GRAPHS, 16), jnp.float32),
        scratch_shapes=[
            pltpu.VMEM((N_GRAPHS, HID), jnp.float32),
            pltpu.VMEM((N_GRAPHS, 8), jnp.float32),
        ],
    )(t, ah, deg16, batch2d, w2aT, w2bT, b2, a2, be2,
      wc1T, bc1, ac, bec, wc2T, bc2)


def _row(v):
    return jnp.reshape(v, (1, -1)).astype(jnp.float32)


def kernel(x, pos_enc, edge_index, batch, params):
    p = params
    f32 = jnp.float32

    h0 = jnp.concatenate([x, pos_enc], axis=-1).astype(f32)
    h0p = jnp.zeros((NPAD, 16), f32).at[:N_NODES, :h0.shape[1]].set(h0)

    src = edge_index[0].astype(jnp.int32)
    dst = edge_index[1].astype(jnp.int32)
    pad_e = EPAD - N_EDGES
    srcp = jnp.concatenate(
        [src, jnp.zeros((pad_e,), jnp.int32)]).reshape(EROWS, CHUNK)
    dstp = jnp.concatenate(
        [dst, jnp.full((pad_e,), 1 << 29, jnp.int32)]).reshape(EROWS, CHUNK)

    batchp = jnp.concatenate(
        [batch.astype(jnp.int32), jnp.full((NPAD - N_NODES,), N_GRAPHS,
                                           jnp.int32)]).reshape(NPAD, 1)

    z64 = jnp.zeros((ZROWS, HID), f32)
    z16 = jnp.zeros((ZROWS, 16), f32)
    one16 = jnp.ones((CHUNK, 16), f32)

    def bn_vecs(pre, k):
        g = p[pre + "_g" + k] * _BN_S
        return (_row(p[pre + "_b" + k]), _row(g), _row(p[pre + "_be" + k]))

    w1c1 = jnp.zeros((16, HID), f32).at[:9, :].set(p["c1_W1"].T)
    b1, a1, be1 = bn_vecs("c1", "1")
    t1 = _tc_dense1(h0p, w1c1, b1, a1, be1)

    deg16, dstloc = _sc_deg(dstp, z16, one16)
    ah1 = _sc_spmv(t1, srcp, dstloc, z64)

    def mid(t, ah, pre_out, pre_in):
        w2 = p[pre_out + "_W2"]
        w2aT = w2[:, :HID].T.astype(f32)
        w2bT = w2[:, HID:].T.astype(f32)
        b2, a2, be2 = bn_vecs(pre_out, "2")
        w1T = p[pre_in + "_W1"].T.astype(f32)
        b1n, a1n, be1n = bn_vecs(pre_in, "1")
        return _tc_mid(t, ah, deg16, w2aT, w2bT, b2, a2, be2,
                       w1T, b1n, a1n, be1n)

    t2 = mid(t1, ah1, "c1", "c2")
    t3 = mid(t2, _sc_spmv(t2, srcp, dstloc, z64), "c2", "c3")
    ah3 = _sc_spmv(t3, srcp, dstloc, z64)

    w2 = p["c3_W2"]
    w2aT = w2[:, :HID].T.astype(f32)
    w2bT = w2[:, HID:].T.astype(f32)
    b2, a2, be2 = bn_vecs("c3", "2")
    wc1T = p["cls_W1"].T.astype(f32)
    bc1 = _row(p["cls_b1"])
    ac = _row(p["cls_g"] * _BN_S)
    bec = _row(p["cls_be"])
    wc2T = jnp.zeros((HID, 16), f32).at[:, :10].set(p["cls_W2"].T)
    bc2 = jnp.full((1, 16), -1e30, f32).at[0, :10].set(p["cls_b2"])

    out = _tc_final(t3, ah3, deg16, batchp, w2aT, w2bT, b2, a2, be2,
                    wc1T, bc1, ac, bec, wc2T, bc2)
    return out[:, :10]
